# Initial kernel scaffold; baseline (speedup 1.0000x reference)
import jax
import jax.numpy as jnp
from jax.experimental import pallas as pl


def kernel(x):
    raise NotImplementedError("write your pallas kernel here")


# speedup vs baseline: 24.0770x; 24.0770x over previous
import math

import jax
import jax.numpy as jnp
from jax.experimental import pallas as pl

_NBINS = 10
_POOL = 8
_H = 512
_W = 512


def _hog_body(x_ref, o_ref):
    x = x_ref[0].astype(jnp.bfloat16).astype(jnp.float32)

    zrow = jnp.zeros((1, _W), jnp.float32)
    x_up = jnp.concatenate([zrow, x[:-1, :]], axis=0)
    x_dn = jnp.concatenate([x[1:, :], zrow], axis=0)
    t = x_up + 2.0 * x + x_dn
    v = x_up - x_dn

    zcol = jnp.zeros((_H, 1), jnp.float32)
    t_l = jnp.concatenate([zcol, t[:, :-1]], axis=1)
    t_r = jnp.concatenate([t[:, 1:], zcol], axis=1)
    g0 = t_l - t_r
    v_l = jnp.concatenate([zcol, v[:, :-1]], axis=1)
    v_r = jnp.concatenate([v[:, 1:], zcol], axis=1)
    g1 = v_l + 2.0 * v + v_r

    mag = jnp.sqrt(g0 * g0 + g1 * g1)

    neg = (g0 < 0.0) | ((g0 == 0.0) & (g1 < 0.0))
    s = jnp.where(neg, -1.0, 1.0)
    a = g0 * s
    b = g1 * s
    fl = jnp.zeros(a.shape, jnp.int32)
    on_edge = a == 0.0
    for j in range(1, _NBINS):
        phi = j * math.pi / _NBINS
        cj = a * math.cos(phi) - b * math.sin(phi)
        fl = fl + (cj >= 0.0).astype(jnp.int32)
        on_edge = on_edge | (cj == 0.0)
    fl = jnp.where((a == 0.0) & (b == 0.0), 0, fl)
    ce = jnp.where(fl == _NBINS - 1, 0, fl + 1)
    ce = jnp.where(on_edge, fl, ce)
    one_minus = 1.0 - mag

    r_ids = jax.lax.broadcasted_iota(jnp.int32, (_H // _POOL, _H), 0)
    c_ids = jax.lax.broadcasted_iota(jnp.int32, (_H // _POOL, _H), 1)
    pool_mat = jnp.where(r_ids == c_ids // _POOL, 1.0, 0.0).astype(jnp.float32)

    scale = 1.0 / (_POOL * _POOL)
    for k in range(_NBINS):
        ck = jnp.where(fl == k, mag, 0.0) + jnp.where(ce == k, one_minus, 0.0)
        rp = jax.lax.dot_general(
            pool_mat, ck, (((1,), (0,)), ((), ())),
            preferred_element_type=jnp.float32)
        out = jax.lax.dot_general(
            rp, pool_mat, (((1,), (1,)), ((), ())),
            preferred_element_type=jnp.float32)
        o_ref[0, k] = out * scale


def kernel(x):
    n = x.shape[0]
    x2 = x.reshape(n, _H, _W)
    out = pl.pallas_call(
        _hog_body,
        grid=(n,),
        in_specs=[pl.BlockSpec((1, _H, _W), lambda b: (b, 0, 0))],
        out_specs=pl.BlockSpec(
            (1, _NBINS, _H // _POOL, _W // _POOL), lambda b: (b, 0, 0, 0)),
        out_shape=jax.ShapeDtypeStruct(
            (n, _NBINS, _H // _POOL, _W // _POOL), jnp.float32),
    )(x2)
    return out
